# baseline (speedup 1.0000x reference)
import functools

import jax
import jax.numpy as jnp
from jax import lax
from jax.experimental import pallas as pl
from jax.experimental.pallas import tpu as pltpu
from jax.experimental.pallas import tpu_sc as plsc

_B, _T, _H, _W, _C = 16, 16, 32, 32, 128
_ROW = _W * _C
_NROWS = _B * _T * _H
_NW = 32
_RPW = _NROWS // _NW
_G = 8
_NG = _RPW // _G
_LANES = 16
_CCHUNKS = _C // _LANES


def _body(x_hbm, t_hbm, h_hbm, w_hbm, out_hbm, tv, hv, wv, xb):
    cid = lax.axis_index("c")
    sid = lax.axis_index("s")
    wid = sid * 2 + cid
    base = wid * _RPW

    pltpu.sync_copy(t_hbm, tv)
    pltpu.sync_copy(h_hbm, hv)
    pltpu.sync_copy(w_hbm, wv)

    def group_body(g, carry):
        row0 = base + g * _G
        start = row0 * _ROW
        pltpu.sync_copy(x_hbm.at[pl.ds(start, _G * _ROW)], xb)

        def row_body(rho, c2):
            r = row0 + rho
            th = r % (_T * _H)
            t = th // _H
            h = th % _H
            tb = t * _C
            hb = h * _C
            regs = [tv[pl.ds(tb + k * _LANES, _LANES)]
                    + hv[pl.ds(hb + k * _LANES, _LANES)]
                    for k in range(_CCHUNKS)]
            rbase = rho * _ROW

            def wblk(j, c3):
                off = rbase + j * _C
                woff = j * _C
                for k in range(_CCHUNKS):
                    o = off + k * _LANES
                    xb[pl.ds(o, _LANES)] = (
                        xb[pl.ds(o, _LANES)]
                        + wv[pl.ds(woff + k * _LANES, _LANES)]
                        + regs[k])
                return c3

            lax.fori_loop(0, _W, wblk, c2)
            return c2

        lax.fori_loop(0, _G, row_body, carry)
        pltpu.sync_copy(xb, out_hbm.at[pl.ds(start, _G * _ROW)])
        return carry

    lax.fori_loop(0, _NG, group_body, 0)


@jax.jit
def _pos_embed_sc(xf, tf, hf, wf):
    mesh = plsc.VectorSubcoreMesh(core_axis_name="c", subcore_axis_name="s")
    f = functools.partial(
        pl.kernel,
        mesh=mesh,
        out_type=jax.ShapeDtypeStruct((_NROWS * _ROW,), jnp.float32),
        scratch_types=[
            pltpu.VMEM((_T * _C,), jnp.float32),
            pltpu.VMEM((_H * _C,), jnp.float32),
            pltpu.VMEM((_W * _C,), jnp.float32),
            pltpu.VMEM((_G * _ROW,), jnp.float32),
        ],
    )(_body)
    return f(xf, tf, hf, wf)


def kernel(x, T_table, H_table, W_table):
    xf = x.reshape(_NROWS * _ROW)
    out = _pos_embed_sc(xf, T_table.reshape(-1), H_table.reshape(-1),
                        W_table.reshape(-1))
    return out.reshape(x.shape)


# speedup vs baseline: 1.2407x; 1.2407x over previous
import functools

import jax
import jax.numpy as jnp
from jax import lax
from jax.experimental import pallas as pl
from jax.experimental.pallas import tpu as pltpu
from jax.experimental.pallas import tpu_sc as plsc

_B, _T, _H, _W, _C = 16, 16, 32, 32, 128
_ROW = _W * _C
_NROWS = _B * _T * _H
_NW = 32
_RPW = _NROWS // _NW
_G = 4
_NG = _RPW // _G
_NBUF = 2
_GSZ = _G * _ROW
_LANES = 16
_CCHUNKS = _C // _LANES


def _body(x_hbm, t_hbm, h_hbm, w_hbm, out_hbm,
          tv, hv, wv, xb0, xb1, ob0, ob1, isem0, isem1, osem0, osem1):
    cid = lax.axis_index("c")
    sid = lax.axis_index("s")
    wid = sid * 2 + cid
    base = wid * _RPW
    xbufs = (xb0, xb1)
    obufs = (ob0, ob1)
    isems = (isem0, isem1)
    osems = (osem0, osem1)

    pltpu.sync_copy(t_hbm, tv)
    pltpu.sync_copy(h_hbm, hv)
    pltpu.sync_copy(w_hbm, wv)

    for b in range(_NBUF):
        start = (base + b * _G) * _ROW
        pltpu.async_copy(x_hbm.at[pl.ds(start, _GSZ)], xbufs[b], isems[b])

    def outer(g0, carry):
        for b in range(_NBUF):
            g = g0 * _NBUF + b
            row0 = base + g * _G
            start = row0 * _ROW
            xb, ob, isem, osem = xbufs[b], obufs[b], isems[b], osems[b]

            pltpu.make_async_copy(x_hbm.at[pl.ds(start, _GSZ)], xb, isem).wait()

            @pl.when(g0 > 0)
            def _():
                pltpu.make_async_copy(ob, out_hbm.at[pl.ds(start, _GSZ)],
                                      osem).wait()

            def row_body(rho, c2):
                r = row0 + rho
                th = r % (_T * _H)
                t = th // _H
                h = th % _H
                tb = t * _C
                hb = h * _C
                regs = [tv[pl.ds(tb + k * _LANES, _LANES)]
                        + hv[pl.ds(hb + k * _LANES, _LANES)]
                        for k in range(_CCHUNKS)]
                rbase = rho * _ROW

                def wblk(j, c3):
                    off = rbase + j * _C
                    woff = j * _C
                    for k in range(_CCHUNKS):
                        o = off + k * _LANES
                        ob[pl.ds(o, _LANES)] = (
                            xb[pl.ds(o, _LANES)]
                            + wv[pl.ds(woff + k * _LANES, _LANES)]
                            + regs[k])
                    return c3

                lax.fori_loop(0, _W, wblk, c2)
                return c2

            lax.fori_loop(0, _G, row_body, 0)

            pltpu.async_copy(ob, out_hbm.at[pl.ds(start, _GSZ)], osem)

            @pl.when(g0 < (_NG // _NBUF) - 1)
            def _():
                nstart = start + _NBUF * _GSZ
                pltpu.async_copy(x_hbm.at[pl.ds(nstart, _GSZ)], xb, isem)
        return carry

    lax.fori_loop(0, _NG // _NBUF, outer, 0)

    for b in range(_NBUF):
        g = _NG - _NBUF + b
        start = (base + g * _G) * _ROW
        pltpu.make_async_copy(obufs[b], out_hbm.at[pl.ds(start, _GSZ)],
                              osems[b]).wait()


@jax.jit
def _pos_embed_sc(xf, tf, hf, wf):
    mesh = plsc.VectorSubcoreMesh(core_axis_name="c", subcore_axis_name="s")
    f = functools.partial(
        pl.kernel,
        mesh=mesh,
        out_type=jax.ShapeDtypeStruct((_NROWS * _ROW,), jnp.float32),
        scratch_types=[
            pltpu.VMEM((_T * _C,), jnp.float32),
            pltpu.VMEM((_H * _C,), jnp.float32),
            pltpu.VMEM((_W * _C,), jnp.float32),
            pltpu.VMEM((_GSZ,), jnp.float32),
            pltpu.VMEM((_GSZ,), jnp.float32),
            pltpu.VMEM((_GSZ,), jnp.float32),
            pltpu.VMEM((_GSZ,), jnp.float32),
            pltpu.SemaphoreType.DMA,
            pltpu.SemaphoreType.DMA,
            pltpu.SemaphoreType.DMA,
            pltpu.SemaphoreType.DMA,
        ],
    )(_body)
    return f(xf, tf, hf, wf)


def kernel(x, T_table, H_table, W_table):
    xf = x.reshape(_NROWS * _ROW)
    out = _pos_embed_sc(xf, T_table.reshape(-1), H_table.reshape(-1),
                        W_table.reshape(-1))
    return out.reshape(x.shape)


# speedup vs baseline: 4.6525x; 3.7499x over previous
import functools

import jax
import jax.numpy as jnp
from jax import lax
from jax.experimental import pallas as pl
from jax.experimental.pallas import tpu as pltpu
from jax.experimental.pallas import tpu_sc as plsc

_B, _T, _H, _W, _C = 16, 16, 32, 32, 128
_ROW = _W * _C
_NROWS = _B * _T * _H
_NW = 32
_RPW = _NROWS // _NW
_G = 4
_NG = _RPW // _G
_NBUF = 2
_GSZ = _G * _ROW
_LANES = 16
_CCHUNKS = _C // _LANES


def _body(x_hbm, t_hbm, h_hbm, w_hbm, out_hbm,
          tv, hv, wv, xb0, xb1, ob0, ob1, isem0, isem1, osem0, osem1):
    cid = lax.axis_index("c")
    sid = lax.axis_index("s")
    wid = sid * 2 + cid
    base = wid * _RPW
    xbufs = (xb0, xb1)
    obufs = (ob0, ob1)
    isems = (isem0, isem1)
    osems = (osem0, osem1)

    pltpu.sync_copy(t_hbm, tv)
    pltpu.sync_copy(h_hbm, hv)
    pltpu.sync_copy(w_hbm, wv)

    for b in range(_NBUF):
        start = (base + b * _G) * _ROW
        pltpu.async_copy(x_hbm.at[pl.ds(start, _GSZ)], xbufs[b], isems[b])

    def outer(g0, carry):
        for b in range(_NBUF):
            g = g0 * _NBUF + b
            row0 = base + g * _G
            start = row0 * _ROW
            xb, ob, isem, osem = xbufs[b], obufs[b], isems[b], osems[b]

            pltpu.make_async_copy(x_hbm.at[pl.ds(start, _GSZ)], xb, isem).wait()

            @pl.when(g0 > 0)
            def _():
                pltpu.make_async_copy(ob, out_hbm.at[pl.ds(start, _GSZ)],
                                      osem).wait()

            threg = []
            for rho in range(_G):
                r = row0 + rho
                th = r % (_T * _H)
                t = th // _H
                h = th % _H
                tb = t * _C
                hb = h * _C
                threg.append([tv[pl.ds(tb + k * _LANES, _LANES)]
                              + hv[pl.ds(hb + k * _LANES, _LANES)]
                              for k in range(_CCHUNKS)])

            def wblk(j, c3):
                jc = j * _C
                for k in range(_CCHUNKS):
                    o = jc + k * _LANES
                    wreg = wv[pl.ds(o, _LANES)]
                    for rho in range(_G):
                        ro = rho * _ROW + o
                        ob[pl.ds(ro, _LANES)] = (
                            xb[pl.ds(ro, _LANES)] + wreg) + threg[rho][k]
                return c3

            lax.fori_loop(0, _W, wblk, 0)

            pltpu.async_copy(ob, out_hbm.at[pl.ds(start, _GSZ)], osem)

            @pl.when(g0 < (_NG // _NBUF) - 1)
            def _():
                nstart = start + _NBUF * _GSZ
                pltpu.async_copy(x_hbm.at[pl.ds(nstart, _GSZ)], xb, isem)
        return carry

    lax.fori_loop(0, _NG // _NBUF, outer, 0)

    for b in range(_NBUF):
        g = _NG - _NBUF + b
        start = (base + g * _G) * _ROW
        pltpu.make_async_copy(obufs[b], out_hbm.at[pl.ds(start, _GSZ)],
                              osems[b]).wait()


@jax.jit
def _pos_embed_sc(xf, tf, hf, wf):
    mesh = plsc.VectorSubcoreMesh(core_axis_name="c", subcore_axis_name="s")
    f = functools.partial(
        pl.kernel,
        mesh=mesh,
        out_type=jax.ShapeDtypeStruct((_NROWS * _ROW,), jnp.float32),
        scratch_types=[
            pltpu.VMEM((_T * _C,), jnp.float32),
            pltpu.VMEM((_H * _C,), jnp.float32),
            pltpu.VMEM((_W * _C,), jnp.float32),
            pltpu.VMEM((_GSZ,), jnp.float32),
            pltpu.VMEM((_GSZ,), jnp.float32),
            pltpu.VMEM((_GSZ,), jnp.float32),
            pltpu.VMEM((_GSZ,), jnp.float32),
            pltpu.SemaphoreType.DMA,
            pltpu.SemaphoreType.DMA,
            pltpu.SemaphoreType.DMA,
            pltpu.SemaphoreType.DMA,
        ],
    )(_body)
    return f(xf, tf, hf, wf)


def kernel(x, T_table, H_table, W_table):
    xf = x.reshape(_NROWS * _ROW)
    out = _pos_embed_sc(xf, T_table.reshape(-1), H_table.reshape(-1),
                        W_table.reshape(-1))
    return out.reshape(x.shape)
